# Initial kernel scaffold; baseline (speedup 1.0000x reference)
import jax
import jax.numpy as jnp
from jax.experimental import pallas as pl


def kernel(weights, edge_src, edge_dst, edge_rel):
    raise NotImplementedError("write your pallas kernel here")


# speedup vs baseline: 51.1124x; 51.1124x over previous
import functools

import jax
import jax.numpy as jnp
from jax import lax
from jax.experimental import pallas as pl
from jax.experimental.pallas import tpu as pltpu
from jax.experimental.pallas import tpu_sc as plsc

N_NODES = 50000
N_REL = 8
N_EDGES = 1600000
EMB = 16

NC = 2
NS = 16
LANES = 16

EDGES_PER_TILE = N_EDGES // (NC * NS)
CHUNK = 2000
N_CHUNKS = EDGES_PER_TILE // CHUNK

HP = 3136
NPAD = NS * HP
DP = 3128
DPAD = NS * DP


def _sc_accumulate(w2d, src, dst, rel):
    mesh = plsc.VectorSubcoreMesh(core_axis_name="c", subcore_axis_name="s")

    @functools.partial(
        pl.kernel,
        mesh=mesh,
        compiler_params=pltpu.CompilerParams(use_tc_tiling_on_sc=False),
        out_type=[
            jax.ShapeDtypeStruct((NC * NPAD, EMB), jnp.float32),
            jax.ShapeDtypeStruct((NC * DPAD,), jnp.float32),
        ],
        scratch_types=[
            pltpu.VMEM((CHUNK,), jnp.int32),
            pltpu.VMEM((CHUNK,), jnp.int32),
            pltpu.VMEM((CHUNK,), jnp.int32),
            pltpu.VMEM((CHUNK,), jnp.int32),
            pltpu.VMEM((CHUNK, EMB), jnp.float32),
            pltpu.VMEM((CHUNK,), jnp.float32),
            pltpu.VMEM((CHUNK,), jnp.float32),
            pltpu.VMEM_SHARED((NPAD, EMB), jnp.float32),
            pltpu.VMEM_SHARED((DPAD,), jnp.float32),
            pltpu.SemaphoreType.DMA,
        ],
    )
    def k(w_hbm, src_hbm, dst_hbm, rel_hbm, h_out, d_out,
          src_v, dst_v, rel_v, col_v, rows_v, ones_v, zero_v, h_sh, d_sh, sem):
        c = lax.axis_index("c")
        s = lax.axis_index("s")
        tile = c * NS + s

        zeros16 = jnp.zeros((LANES,), jnp.float32)
        ones16 = jnp.ones((LANES,), jnp.float32)

        def init_body(i, _):
            rows_v[i, :] = zeros16
            return _
        lax.fori_loop(0, CHUNK, init_body, None)

        def init_flat(i, _):
            ones_v[pl.ds(i * LANES, LANES)] = ones16
            zero_v[pl.ds(i * LANES, LANES)] = zeros16
            return _
        lax.fori_loop(0, CHUNK // LANES, init_flat, None)

        hbase = s * HP
        pltpu.sync_copy(rows_v, h_sh.at[pl.ds(hbase, CHUNK)])
        pltpu.sync_copy(rows_v.at[pl.ds(0, HP - CHUNK)],
                        h_sh.at[pl.ds(hbase + CHUNK, HP - CHUNK)])
        dbase = s * DP
        pltpu.sync_copy(zero_v, d_sh.at[pl.ds(dbase, CHUNK)])
        pltpu.sync_copy(zero_v.at[pl.ds(0, DP - CHUNK)],
                        d_sh.at[pl.ds(dbase + CHUNK, DP - CHUNK)])

        plsc.subcore_barrier()

        ebase = tile * EDGES_PER_TILE

        def chunk_body(kk, _):
            off = ebase + kk * CHUNK
            pltpu.sync_copy(src_hbm.at[pl.ds(off, CHUNK)], src_v)
            pltpu.sync_copy(dst_hbm.at[pl.ds(off, CHUNK)], dst_v)
            pltpu.sync_copy(rel_hbm.at[pl.ds(off, CHUNK)], rel_v)

            def col_body(i, _):
                sl = pl.ds(i * LANES, LANES)
                col_v[sl] = rel_v[sl] * jnp.int32(N_NODES) + dst_v[sl]
                return _
            lax.fori_loop(0, CHUNK // LANES, col_body, None)

            pltpu.async_copy(w_hbm.at[col_v], rows_v, sem).wait()
            pltpu.sync_copy(rows_v, h_sh.at[src_v], add=True)
            pltpu.sync_copy(ones_v, d_sh.at[src_v], add=True)
            return _
        lax.fori_loop(0, N_CHUNKS, chunk_body, None)

        plsc.subcore_barrier()

        pltpu.sync_copy(h_sh.at[pl.ds(hbase, HP)],
                        h_out.at[pl.ds(c * NPAD + hbase, HP)])
        pltpu.sync_copy(d_sh.at[pl.ds(dbase, DP)],
                        d_out.at[pl.ds(c * DPAD + dbase, DP)])

    return k(w2d, src, dst, rel)


def _combine(p0, p1, d0, d1):
    def body(p0_ref, p1_ref, d0_ref, d1_ref, o_ref):
        deg = d0_ref[...] + d1_ref[...]
        scale = 1.0 / jnp.maximum(deg, 1.0)
        o_ref[...] = (p0_ref[...] + p1_ref[...]) * scale

    BR = 5000
    return pl.pallas_call(
        body,
        grid=(N_NODES // BR,),
        in_specs=[
            pl.BlockSpec((BR, EMB), lambda i: (i, 0)),
            pl.BlockSpec((BR, EMB), lambda i: (i, 0)),
            pl.BlockSpec((BR, 1), lambda i: (i, 0)),
            pl.BlockSpec((BR, 1), lambda i: (i, 0)),
        ],
        out_specs=pl.BlockSpec((BR, EMB), lambda i: (i, 0)),
        out_shape=jax.ShapeDtypeStruct((N_NODES, EMB), jnp.float32),
    )(p0, p1, d0, d1)


def kernel(weights, edge_src, edge_dst, edge_rel):
    w2d = weights.reshape(N_REL * N_NODES, EMB)
    src = edge_src.astype(jnp.int32)
    dst = edge_dst.astype(jnp.int32)
    rel = edge_rel.astype(jnp.int32)
    h_part, d_part = _sc_accumulate(w2d, src, dst, rel)
    p0 = h_part[:N_NODES]
    p1 = h_part[NPAD:NPAD + N_NODES]
    d0 = d_part[:N_NODES].reshape(N_NODES, 1)
    d1 = d_part[DPAD:DPAD + N_NODES].reshape(N_NODES, 1)
    return _combine(p0, p1, d0, d1)
